# baseline (speedup 1.0000x reference)
import functools

import jax
import jax.numpy as jnp
from jax import lax
from jax.experimental import pallas as pl
from jax.experimental.pallas import tpu as pltpu
from jax.experimental.pallas import tpu_sc as plsc

N_NODES = 10000
N_EDGES = 320000
N_REL = 4

NC = 2
NS = 16
NW = NC * NS
EPW = N_EDGES // NW
CHUNK = 80
NCH = EPW // CHUNK
CNTP = 40960
CSTR = CNTP // NS
NPAD = 10240
NRPT = NPAD // NS
ZR = 128

_mesh = plsc.VectorSubcoreMesh(core_axis_name="c", subcore_axis_name="s")
_sc_params = pltpu.CompilerParams(needs_layout_passes=False)


def _mm_l1(x, w_aug, b1, cnt):
    BN = 1000
    NB = N_NODES // BN

    def body(x_ref, w_ref, b_ref, c_ref, o_ref, inv_ref):
        acc = jnp.dot(x_ref[...], w_ref[0], preferred_element_type=jnp.float32)
        r = pl.program_id(0)
        i = pl.program_id(1)
        o_ref[0] = jnp.where(r == N_REL, acc + b_ref[...], acc)

        @pl.when(jnp.logical_and(r == 0, i == 0))
        def _():
            inv_ref[...] = 1.0 / jnp.maximum(c_ref[0] + c_ref[1], 1.0)

    out, inv = pl.pallas_call(
        body,
        grid=(N_REL + 1, NB),
        in_specs=[
            pl.BlockSpec((BN, 128), lambda r, i: (i, 0)),
            pl.BlockSpec((1, 128, 128), lambda r, i: (r, 0, 0)),
            pl.BlockSpec((1, 128), lambda r, i: (0, 0)),
            pl.BlockSpec((2, CNTP // 128, 128), lambda r, i: (0, 0, 0)),
        ],
        out_specs=[pl.BlockSpec((1, BN, 128), lambda r, i: (r, i, 0)),
                   pl.BlockSpec((CNTP // 128, 128), lambda r, i: (0, 0))],
        out_shape=[
            jax.ShapeDtypeStruct((N_REL + 1, N_NODES, 128), jnp.float32),
            jax.ShapeDtypeStruct((CNTP // 128, 128), jnp.float32),
        ],
    )(x, w_aug, b1, cnt.reshape(2, CNTP // 128, 128))
    return out, inv.reshape(CNTP)


def _mm_l2(h1, m1, w2mask, root2p, b2p):
    BN = 1000
    NB = N_NODES // BN

    def body(r_ref, m0_ref, m1_ref, w_ref, rt_ref, b_ref, t_ref, r2_ref):
        a = jnp.maximum(r_ref[0] + m0_ref[0] + m1_ref[0], 0.0)
        t_ref[0] = jnp.dot(a, w_ref[0], preferred_element_type=jnp.float32)
        r2_ref[...] = (jnp.dot(a, rt_ref[...], preferred_element_type=jnp.float32)
                       + b_ref[...])

    return pl.pallas_call(
        body,
        grid=(N_REL, NB),
        in_specs=[
            pl.BlockSpec((1, BN, 128), lambda r, i: (N_REL, i, 0)),
            pl.BlockSpec((1, BN, 128), lambda r, i: (0, i, 0)),
            pl.BlockSpec((1, BN, 128), lambda r, i: (1, i, 0)),
            pl.BlockSpec((1, 128, 128), lambda r, i: (r, 0, 0)),
            pl.BlockSpec((128, 16), lambda r, i: (0, 0)),
            pl.BlockSpec((1, 16), lambda r, i: (0, 0)),
        ],
        out_specs=[pl.BlockSpec((1, BN, 128), lambda r, i: (r, i, 0)),
                   pl.BlockSpec((BN, 16), lambda r, i: (i, 0))],
        out_shape=[
            jax.ShapeDtypeStruct((N_REL, N_NODES, 128), jnp.float32),
            jax.ShapeDtypeStruct((N_NODES, 16), jnp.float32),
        ],
    )(h1, m1, m1, w2mask, root2p, b2p)


def _final_add(r2, m2):
    BN = 1000
    NB = N_NODES // BN

    def body(r_ref, m0_ref, m1_ref, o_ref):
        m = m0_ref[0] + m1_ref[0]
        s = (m[:, 0:2] + m[:, 32:34] + m[:, 64:66] + m[:, 96:98])
        o_ref[...] = r_ref[...] + jnp.concatenate(
            [s, jnp.zeros((BN, 14), jnp.float32)], axis=1)

    spec16 = pl.BlockSpec((BN, 16), lambda i: (i, 0))
    return pl.pallas_call(
        body,
        grid=(NB,),
        in_specs=[spec16,
                  pl.BlockSpec((1, BN, 128), lambda i: (0, i, 0)),
                  pl.BlockSpec((1, BN, 128), lambda i: (1, i, 0))],
        out_specs=spec16,
        out_shape=jax.ShapeDtypeStruct((N_NODES, 16), jnp.float32),
    )(r2, m2, m2)


@functools.partial(
    pl.kernel,
    out_type=(
        jax.ShapeDtypeStruct((N_EDGES,), jnp.int32),
        jax.ShapeDtypeStruct((NC, CNTP), jnp.float32),
    ),
    mesh=_mesh,
    compiler_params=_sc_params,
    scratch_types=[
        pltpu.VMEM((CHUNK,), jnp.int32),
        pltpu.VMEM((CHUNK,), jnp.int32),
        pltpu.VMEM((CHUNK,), jnp.int32),
        pltpu.VMEM((CHUNK,), jnp.int32),
        pltpu.VMEM((CHUNK,), jnp.int32),
        pltpu.VMEM((CHUNK,), jnp.int32),
        pltpu.VMEM((CHUNK,), jnp.int32),
        pltpu.VMEM((CHUNK,), jnp.int32),
        pltpu.VMEM((CHUNK,), jnp.int32),
        pltpu.VMEM((CHUNK,), jnp.int32),
        pltpu.VMEM((CHUNK,), jnp.float32),
        pltpu.VMEM((CSTR,), jnp.float32),
        pltpu.VMEM_SHARED((CNTP,), jnp.float32),
        pltpu.SemaphoreType.DMA,
        pltpu.SemaphoreType.DMA,
        pltpu.SemaphoreType.DMA,
        pltpu.SemaphoreType.DMA,
        pltpu.SemaphoreType.DMA,
        pltpu.SemaphoreType.DMA,
    ],
)
def _count_kernel(src_hbm, dst_hbm, typ_hbm, gidx_hbm, cnt_hbm,
                  sbuf0, sbuf1, dbuf0, dbuf1, tbuf0, tbuf1,
                  gbuf0, gbuf1, wibuf0, wibuf1, ones, zbuf, cnt_sh,
                  lsem0, lsem1, stsem0, stsem1, scsem0, scsem1):
    sbuf = (sbuf0, sbuf1)
    dbuf = (dbuf0, dbuf1)
    tbuf = (tbuf0, tbuf1)
    gbuf = (gbuf0, gbuf1)
    wibuf = (wibuf0, wibuf1)
    lsem = (lsem0, lsem1)
    stsem = (stsem0, stsem1)
    scsem = (scsem0, scsem1)
    cid = lax.axis_index("c")
    sid = lax.axis_index("s")
    wid = cid * NS + sid
    ebase = wid * EPW
    zero16 = jnp.zeros((16,), jnp.float32)
    one16 = jnp.ones((16,), jnp.float32)

    def zfill(i, _):
        zbuf[pl.ds(i * 16, 16)] = zero16
        return 0
    lax.fori_loop(0, CSTR // 16, zfill, 0)
    for g in range(CHUNK // 16):
        ones[pl.ds(g * 16, 16)] = one16
    pltpu.sync_copy(zbuf, cnt_sh.at[pl.ds(sid * CSTR, CSTR)])
    pltpu.async_copy(src_hbm.at[pl.ds(ebase, CHUNK)], sbuf0, lsem0)
    pltpu.async_copy(dst_hbm.at[pl.ds(ebase, CHUNK)], dbuf0, lsem0)
    pltpu.async_copy(typ_hbm.at[pl.ds(ebase, CHUNK)], tbuf0, lsem0)
    plsc.subcore_barrier()

    def step(c, b):
        nb = 1 - b
        base = ebase + c * CHUNK
        base_n = base + CHUNK
        pltpu.make_async_copy(src_hbm.at[pl.ds(base, CHUNK)], sbuf[b],
                              lsem[b]).wait()
        pltpu.make_async_copy(dst_hbm.at[pl.ds(base, CHUNK)], dbuf[b],
                              lsem[b]).wait()
        pltpu.make_async_copy(typ_hbm.at[pl.ds(base, CHUNK)], tbuf[b],
                              lsem[b]).wait()

        @pl.when(c + 1 < NCH)
        def _():
            pltpu.async_copy(src_hbm.at[pl.ds(base_n, CHUNK)], sbuf[nb],
                             lsem[nb])
            pltpu.async_copy(dst_hbm.at[pl.ds(base_n, CHUNK)], dbuf[nb],
                             lsem[nb])
            pltpu.async_copy(typ_hbm.at[pl.ds(base_n, CHUNK)], tbuf[nb],
                             lsem[nb])
        @pl.when(c >= 2)
        def _():
            base_p = base - 2 * CHUNK
            pltpu.make_async_copy(gbuf[b], gidx_hbm.at[pl.ds(base_p, CHUNK)],
                                  stsem[b]).wait()
            pltpu.make_async_copy(ones, cnt_sh.at[wibuf[b]], scsem[b]).wait()
        for g in range(CHUNK // 16):
            sl = pl.ds(g * 16, 16)
            t = tbuf[b][sl]
            tn = t * N_NODES
            gbuf[b][sl] = ((tn + sbuf[b][sl])
                           | lax.shift_left(dbuf[b][sl], 16)
                           | lax.shift_left(t, 30))
            wibuf[b][sl] = tn + dbuf[b][sl]
        pltpu.async_copy(gbuf[b], gidx_hbm.at[pl.ds(base, CHUNK)], stsem[b])
        pltpu.async_copy(ones, cnt_sh.at[wibuf[b]], scsem[b], add=True)

    step(0, 0)

    def pair(k, _):
        step(2 * k + 1, 1)
        step(2 * k + 2, 0)
        return 0
    lax.fori_loop(0, (NCH - 1) // 2, pair, 0)
    for c in (NCH - 2, NCH - 1):
        b = c % 2
        base = ebase + c * CHUNK
        pltpu.make_async_copy(gbuf[b], gidx_hbm.at[pl.ds(base, CHUNK)],
                              stsem[b]).wait()
        pltpu.make_async_copy(ones, cnt_sh.at[wibuf[b]], scsem[b]).wait()
    plsc.subcore_barrier()
    pltpu.sync_copy(cnt_sh.at[pl.ds(sid * CSTR, CSTR)],
                    cnt_hbm.at[cid, pl.ds(sid * CSTR, CSTR)])


D_MSG = 128

_MSG_SCRATCH = [
    pltpu.VMEM((EPW,), jnp.int32),
    pltpu.VMEM((EPW + 16,), jnp.float32),
    pltpu.VMEM((CHUNK,), jnp.int32),
    pltpu.VMEM((CHUNK,), jnp.int32),
    pltpu.VMEM((CHUNK,), jnp.int32),
    pltpu.VMEM((CHUNK,), jnp.int32),
    pltpu.VMEM((CHUNK,), jnp.int32),
    pltpu.VMEM((CHUNK,), jnp.int32),
    pltpu.VMEM((CHUNK + 16,), jnp.float32),
    pltpu.VMEM((CHUNK + 16,), jnp.float32),
    pltpu.VMEM((CHUNK, D_MSG), jnp.float32),
    pltpu.VMEM((CHUNK, D_MSG), jnp.float32),
    pltpu.VMEM_SHARED((NPAD, D_MSG), jnp.float32),
] + [pltpu.SemaphoreType.DMA] * 8

_M16 = (1 << 16) - 1
_M14 = (1 << 14) - 1


def _make_msg_kernel(emit_w):
    D = D_MSG
    if emit_w:
        out_type = (jax.ShapeDtypeStruct((NC, NPAD, D), jnp.float32),
                    jax.ShapeDtypeStruct((N_EDGES,), jnp.float32))
    else:
        out_type = jax.ShapeDtypeStruct((NC, NPAD, D), jnp.float32)

    @functools.partial(
        pl.kernel,
        out_type=out_type,
        mesh=_mesh,
        compiler_params=_sc_params,
        scratch_types=_MSG_SCRATCH,
    )
    def msg_kernel(*args):
        if emit_w:
            (tbl_hbm, pk_hbm, inv_hbm, out_hbm, w_hbm, *rest) = args
        else:
            (tbl_hbm, pk_hbm, w_hbm, out_hbm, *rest) = args
            inv_hbm = None
        (pall, wall, gbuf0, gbuf1, dbuf0, dbuf1, wibuf0, wibuf1,
         wbuf0, wbuf1, rows0, rows1, acc,
         gsem0, gsem1, ssem0, ssem1, wsem0, wsem1, wssem0, wssem1) = rest
        gbuf = (gbuf0, gbuf1)
        dbuf = (dbuf0, dbuf1)
        wibuf = (wibuf0, wibuf1)
        wbuf = (wbuf0, wbuf1)
        rows = (rows0, rows1)
        gsem = (gsem0, gsem1)
        ssem = (ssem0, ssem1)
        wsem = (wsem0, wsem1)
        wssem = (wssem0, wssem1)
        cid = lax.axis_index("c")
        sid = lax.axis_index("s")
        wid = cid * NS + sid
        ebase = wid * EPW
        zero16 = jnp.zeros((16,), jnp.float32)

        def zfill(i, _):
            for v in range(D // 16):
                rows1[i, pl.ds(v * 16, 16)] = zero16
            return 0
        lax.fori_loop(0, CHUNK, zfill, 0)
        for t in range(NRPT // CHUNK):
            pltpu.sync_copy(rows1,
                            acc.at[pl.ds(sid * NRPT + t * CHUNK, CHUNK), :])

        pltpu.sync_copy(pk_hbm.at[pl.ds(ebase, EPW)], pall)
        if not emit_w:
            pltpu.sync_copy(w_hbm.at[pl.ds(ebase, EPW)],
                            wall.at[pl.ds(0, EPW)])

        def unpack(c, s):
            off = c * CHUNK
            for g in range(CHUNK // 16):
                sl = pl.ds(g * 16, 16)
                pv = pall[pl.ds(off + g * 16, 16)]
                gbuf[s][sl] = pv & _M16
                d = lax.shift_right_logical(pv, 16) & _M14
                dbuf[s][sl] = d
                if emit_w:
                    wibuf[s][sl] = (lax.shift_right_logical(pv, 30) * N_NODES
                                    + d)

        unpack(0, 0)
        pltpu.async_copy(tbl_hbm.at[gbuf0], rows0, gsem0)
        plsc.subcore_barrier()

        def step(c, b, first):
            nb = 1 - b
            pltpu.make_async_copy(tbl_hbm.at[gbuf[b]], rows[b], gsem[b]).wait()
            if emit_w:
                if not first:
                    @pl.when(c >= 2)
                    def _():
                        pltpu.make_async_copy(
                            wbuf[b].at[pl.ds(0, CHUNK)],
                            w_hbm.at[pl.ds(ebase + (c - 2) * CHUNK, CHUNK)],
                            wssem[b]).wait()
                wdesc = pltpu.async_copy(inv_hbm.at[wibuf[b]],
                                         wbuf[b].at[pl.ds(0, CHUNK)], wsem[b])
            if not first:
                pltpu.make_async_copy(rows[nb], acc.at[dbuf[nb]],
                                      ssem[nb]).wait()

            @pl.when(c + 1 < NCH)
            def _():
                unpack(c + 1, nb)
                pltpu.async_copy(tbl_hbm.at[gbuf[nb]], rows[nb], gsem[nb])
            if emit_w:
                wdesc.wait()

                @plsc.parallel_loop(0, CHUNK, 1, unroll=4)
                def scale(j):
                    w = wbuf[b][pl.ds(j, 16)][0]
                    for v in range(D // 16):
                        s2 = pl.ds(v * 16, 16)
                        rows[b][j, s2] = rows[b][j, s2] * w
                pltpu.async_copy(wbuf[b].at[pl.ds(0, CHUNK)],
                                 w_hbm.at[pl.ds(ebase + c * CHUNK, CHUNK)],
                                 wssem[b])
            else:
                off = c * CHUNK

                @plsc.parallel_loop(0, CHUNK, 1, unroll=4)
                def scale(j):
                    w = wall[pl.ds(off + j, 16)][0]
                    for v in range(D // 16):
                        s2 = pl.ds(v * 16, 16)
                        rows[b][j, s2] = rows[b][j, s2] * w
            pltpu.async_copy(rows[b], acc.at[dbuf[b]], ssem[b], add=True)

        step(0, 0, True)

        def pair(k, _):
            step(2 * k + 1, 1, False)
            step(2 * k + 2, 0, False)
            return 0
        lax.fori_loop(0, (NCH - 1) // 2, pair, 0)
        fb = (NCH - 1) % 2
        pltpu.make_async_copy(rows[fb], acc.at[dbuf[fb]], ssem[fb]).wait()
        if emit_w:
            for c in (NCH - 2, NCH - 1):
                bb = c % 2
                pltpu.make_async_copy(
                    wbuf[bb].at[pl.ds(0, CHUNK)],
                    w_hbm.at[pl.ds(ebase + c * CHUNK, CHUNK)],
                    wssem[bb]).wait()
        plsc.subcore_barrier()
        pltpu.sync_copy(acc.at[pl.ds(sid * NRPT, NRPT), :],
                        out_hbm.at[cid, pl.ds(sid * NRPT, NRPT), :])
    return msg_kernel


_msg_emit = _make_msg_kernel(True)
_msg_load = _make_msg_kernel(False)


def kernel(x, edge_index, edge_type, W1, root1, b1, W2, root2, b2):
    x = x.astype(jnp.float32)
    ei = edge_index.astype(jnp.int32)
    et = edge_type.astype(jnp.int32)
    src = ei[0]
    dst = ei[1]

    packed, cnt = _count_kernel(src, dst, et)
    w1_aug = jnp.concatenate([W1, root1[None]], axis=0)
    h1, inv = _mm_l1(x, w1_aug, b1.reshape(1, 128), cnt)

    m1, w_edge = _msg_emit(h1.reshape((N_REL + 1) * N_NODES, 128), packed,
                           inv)

    w2mask = jnp.zeros((N_REL, 128, 128), jnp.float32)
    for r in range(N_REL):
        w2mask = w2mask.at[r, :, 32 * r:32 * r + 2].set(W2[r])
    root2p = jnp.zeros((128, 16), jnp.float32).at[:, :2].set(root2)
    b2p = jnp.zeros((1, 16), jnp.float32).at[0, :2].set(b2)

    t2, r2 = _mm_l2(h1, m1, w2mask, root2p, b2p)

    m2 = _msg_load(t2.reshape(N_REL * N_NODES, 128), packed, w_edge)

    out16 = _final_add(r2, m2)
    return out16[:, :2]


# speedup vs baseline: 1.0674x; 1.0674x over previous
import functools

import jax
import jax.numpy as jnp
from jax import lax
from jax.experimental import pallas as pl
from jax.experimental.pallas import tpu as pltpu
from jax.experimental.pallas import tpu_sc as plsc

N_NODES = 10000
N_EDGES = 320000
N_REL = 4

NC = 2
NS = 16
NW = NC * NS
EPW = N_EDGES // NW
CHUNK = 80
NCH = EPW // CHUNK
CNTP = 40960
CSTR = CNTP // NS
NPAD = 10240
NRPT = NPAD // NS
ZR = 128

_mesh = plsc.VectorSubcoreMesh(core_axis_name="c", subcore_axis_name="s")
_sc_params = pltpu.CompilerParams(needs_layout_passes=False)


def _mm_l1(x, w_aug, b1):
    BN = 1000
    NB = N_NODES // BN

    def body(x_ref, w_ref, b_ref, o_ref):
        acc = jnp.dot(x_ref[...], w_ref[0], preferred_element_type=jnp.float32)
        r = pl.program_id(0)
        o_ref[0] = jnp.where(r == N_REL, acc + b_ref[...], acc)

    return pl.pallas_call(
        body,
        grid=(N_REL + 1, NB),
        in_specs=[
            pl.BlockSpec((BN, 128), lambda r, i: (i, 0)),
            pl.BlockSpec((1, 128, 128), lambda r, i: (r, 0, 0)),
            pl.BlockSpec((1, 128), lambda r, i: (0, 0)),
        ],
        out_specs=pl.BlockSpec((1, BN, 128), lambda r, i: (r, i, 0)),
        out_shape=jax.ShapeDtypeStruct((N_REL + 1, N_NODES, 128), jnp.float32),
    )(x, w_aug, b1)


def _inv_counts(cnt):
    c3 = cnt.reshape(2, CNTP // 128, 128)

    def body(c_ref, o_ref):
        o_ref[...] = 1.0 / jnp.maximum(c_ref[0] + c_ref[1], 1.0)

    out = pl.pallas_call(
        body,
        out_shape=jax.ShapeDtypeStruct((CNTP // 128, 128), jnp.float32),
    )(c3)
    return out.reshape(CNTP)


def _mm_l2(h1, m1, w2mask, root2p, b2p):
    BN = 1000
    NB = N_NODES // BN

    def body(r_ref, m0_ref, m1_ref, w_ref, rt_ref, b_ref, t_ref, r2_ref):
        a = jnp.maximum(r_ref[0] + m0_ref[0] + m1_ref[0], 0.0)
        t_ref[0] = jnp.dot(a, w_ref[0], preferred_element_type=jnp.float32)
        r2_ref[...] = (jnp.dot(a, rt_ref[...], preferred_element_type=jnp.float32)
                       + b_ref[...])

    return pl.pallas_call(
        body,
        grid=(N_REL, NB),
        in_specs=[
            pl.BlockSpec((1, BN, 128), lambda r, i: (N_REL, i, 0)),
            pl.BlockSpec((1, BN, 128), lambda r, i: (0, i, 0)),
            pl.BlockSpec((1, BN, 128), lambda r, i: (1, i, 0)),
            pl.BlockSpec((1, 128, 128), lambda r, i: (r, 0, 0)),
            pl.BlockSpec((128, 16), lambda r, i: (0, 0)),
            pl.BlockSpec((1, 16), lambda r, i: (0, 0)),
        ],
        out_specs=[pl.BlockSpec((1, BN, 128), lambda r, i: (r, i, 0)),
                   pl.BlockSpec((BN, 16), lambda r, i: (i, 0))],
        out_shape=[
            jax.ShapeDtypeStruct((N_REL, N_NODES, 128), jnp.float32),
            jax.ShapeDtypeStruct((N_NODES, 16), jnp.float32),
        ],
    )(h1, m1, m1, w2mask, root2p, b2p)


def _final_add(r2, m2):
    BN = 1000
    NB = N_NODES // BN

    def body(r_ref, m0_ref, m1_ref, o_ref):
        m = m0_ref[0] + m1_ref[0]
        s = (m[:, 0:2] + m[:, 32:34] + m[:, 64:66] + m[:, 96:98])
        o_ref[...] = r_ref[...] + jnp.concatenate(
            [s, jnp.zeros((BN, 14), jnp.float32)], axis=1)

    spec16 = pl.BlockSpec((BN, 16), lambda i: (i, 0))
    return pl.pallas_call(
        body,
        grid=(NB,),
        in_specs=[spec16,
                  pl.BlockSpec((1, BN, 128), lambda i: (0, i, 0)),
                  pl.BlockSpec((1, BN, 128), lambda i: (1, i, 0))],
        out_specs=spec16,
        out_shape=jax.ShapeDtypeStruct((N_NODES, 16), jnp.float32),
    )(r2, m2, m2)


@functools.partial(
    pl.kernel,
    out_type=(
        jax.ShapeDtypeStruct((N_EDGES,), jnp.int32),
        jax.ShapeDtypeStruct((NC, CNTP), jnp.float32),
    ),
    mesh=_mesh,
    compiler_params=_sc_params,
    scratch_types=[
        pltpu.VMEM((CHUNK,), jnp.int32),
        pltpu.VMEM((CHUNK,), jnp.int32),
        pltpu.VMEM((CHUNK,), jnp.int32),
        pltpu.VMEM((CHUNK,), jnp.int32),
        pltpu.VMEM((CHUNK,), jnp.int32),
        pltpu.VMEM((CHUNK,), jnp.int32),
        pltpu.VMEM((CHUNK,), jnp.int32),
        pltpu.VMEM((CHUNK,), jnp.int32),
        pltpu.VMEM((CHUNK,), jnp.int32),
        pltpu.VMEM((CHUNK,), jnp.int32),
        pltpu.VMEM((CHUNK,), jnp.float32),
        pltpu.VMEM((CSTR,), jnp.float32),
        pltpu.VMEM_SHARED((CNTP,), jnp.float32),
        pltpu.SemaphoreType.DMA,
        pltpu.SemaphoreType.DMA,
        pltpu.SemaphoreType.DMA,
        pltpu.SemaphoreType.DMA,
        pltpu.SemaphoreType.DMA,
        pltpu.SemaphoreType.DMA,
    ],
)
def _count_kernel(src_hbm, dst_hbm, typ_hbm, gidx_hbm, cnt_hbm,
                  sbuf0, sbuf1, dbuf0, dbuf1, tbuf0, tbuf1,
                  gbuf0, gbuf1, wibuf0, wibuf1, ones, zbuf, cnt_sh,
                  lsem0, lsem1, stsem0, stsem1, scsem0, scsem1):
    sbuf = (sbuf0, sbuf1)
    dbuf = (dbuf0, dbuf1)
    tbuf = (tbuf0, tbuf1)
    gbuf = (gbuf0, gbuf1)
    wibuf = (wibuf0, wibuf1)
    lsem = (lsem0, lsem1)
    stsem = (stsem0, stsem1)
    scsem = (scsem0, scsem1)
    cid = lax.axis_index("c")
    sid = lax.axis_index("s")
    wid = cid * NS + sid
    ebase = wid * EPW
    zero16 = jnp.zeros((16,), jnp.float32)
    one16 = jnp.ones((16,), jnp.float32)

    def zfill(i, _):
        zbuf[pl.ds(i * 16, 16)] = zero16
        return 0
    lax.fori_loop(0, CSTR // 16, zfill, 0)
    for g in range(CHUNK // 16):
        ones[pl.ds(g * 16, 16)] = one16
    pltpu.sync_copy(zbuf, cnt_sh.at[pl.ds(sid * CSTR, CSTR)])
    pltpu.async_copy(src_hbm.at[pl.ds(ebase, CHUNK)], sbuf0, lsem0)
    pltpu.async_copy(dst_hbm.at[pl.ds(ebase, CHUNK)], dbuf0, lsem0)
    pltpu.async_copy(typ_hbm.at[pl.ds(ebase, CHUNK)], tbuf0, lsem0)
    plsc.subcore_barrier()

    def step(c, b):
        nb = 1 - b
        base = ebase + c * CHUNK
        base_n = base + CHUNK
        pltpu.make_async_copy(src_hbm.at[pl.ds(base, CHUNK)], sbuf[b],
                              lsem[b]).wait()
        pltpu.make_async_copy(dst_hbm.at[pl.ds(base, CHUNK)], dbuf[b],
                              lsem[b]).wait()
        pltpu.make_async_copy(typ_hbm.at[pl.ds(base, CHUNK)], tbuf[b],
                              lsem[b]).wait()

        @pl.when(c + 1 < NCH)
        def _():
            pltpu.async_copy(src_hbm.at[pl.ds(base_n, CHUNK)], sbuf[nb],
                             lsem[nb])
            pltpu.async_copy(dst_hbm.at[pl.ds(base_n, CHUNK)], dbuf[nb],
                             lsem[nb])
            pltpu.async_copy(typ_hbm.at[pl.ds(base_n, CHUNK)], tbuf[nb],
                             lsem[nb])
        @pl.when(c >= 2)
        def _():
            base_p = base - 2 * CHUNK
            pltpu.make_async_copy(gbuf[b], gidx_hbm.at[pl.ds(base_p, CHUNK)],
                                  stsem[b]).wait()
            pltpu.make_async_copy(ones, cnt_sh.at[wibuf[b]], scsem[b]).wait()
        for g in range(CHUNK // 16):
            sl = pl.ds(g * 16, 16)
            t = tbuf[b][sl]
            tn = t * N_NODES
            gbuf[b][sl] = ((tn + sbuf[b][sl])
                           | lax.shift_left(dbuf[b][sl], 16)
                           | lax.shift_left(t, 30))
            wibuf[b][sl] = tn + dbuf[b][sl]
        pltpu.async_copy(gbuf[b], gidx_hbm.at[pl.ds(base, CHUNK)], stsem[b])
        pltpu.async_copy(ones, cnt_sh.at[wibuf[b]], scsem[b], add=True)

    step(0, 0)

    def pair(k, _):
        step(2 * k + 1, 1)
        step(2 * k + 2, 0)
        return 0
    lax.fori_loop(0, (NCH - 1) // 2, pair, 0)
    for c in (NCH - 2, NCH - 1):
        b = c % 2
        base = ebase + c * CHUNK
        pltpu.make_async_copy(gbuf[b], gidx_hbm.at[pl.ds(base, CHUNK)],
                              stsem[b]).wait()
        pltpu.make_async_copy(ones, cnt_sh.at[wibuf[b]], scsem[b]).wait()
    plsc.subcore_barrier()
    pltpu.sync_copy(cnt_sh.at[pl.ds(sid * CSTR, CSTR)],
                    cnt_hbm.at[cid, pl.ds(sid * CSTR, CSTR)])


D_MSG = 128

_MSG_SCRATCH = [
    pltpu.VMEM((EPW,), jnp.int32),
    pltpu.VMEM((EPW + 16,), jnp.float32),
    pltpu.VMEM((CHUNK,), jnp.int32),
    pltpu.VMEM((CHUNK,), jnp.int32),
    pltpu.VMEM((CHUNK,), jnp.int32),
    pltpu.VMEM((CHUNK,), jnp.int32),
    pltpu.VMEM((CHUNK,), jnp.int32),
    pltpu.VMEM((CHUNK,), jnp.int32),
    pltpu.VMEM((CHUNK + 16,), jnp.float32),
    pltpu.VMEM((CHUNK + 16,), jnp.float32),
    pltpu.VMEM((CHUNK, D_MSG), jnp.float32),
    pltpu.VMEM((CHUNK, D_MSG), jnp.float32),
    pltpu.VMEM_SHARED((NPAD, D_MSG), jnp.float32),
] + [pltpu.SemaphoreType.DMA] * 8

_M16 = (1 << 16) - 1
_M14 = (1 << 14) - 1


def _make_msg_kernel(emit_w):
    D = D_MSG
    if emit_w:
        out_type = (jax.ShapeDtypeStruct((NC, NPAD, D), jnp.float32),
                    jax.ShapeDtypeStruct((N_EDGES,), jnp.float32))
    else:
        out_type = jax.ShapeDtypeStruct((NC, NPAD, D), jnp.float32)

    @functools.partial(
        pl.kernel,
        out_type=out_type,
        mesh=_mesh,
        compiler_params=_sc_params,
        scratch_types=_MSG_SCRATCH,
    )
    def msg_kernel(*args):
        if emit_w:
            (tbl_hbm, pk_hbm, inv_hbm, out_hbm, w_hbm, *rest) = args
        else:
            (tbl_hbm, pk_hbm, w_hbm, out_hbm, *rest) = args
            inv_hbm = None
        (pall, wall, gbuf0, gbuf1, dbuf0, dbuf1, wibuf0, wibuf1,
         wbuf0, wbuf1, rows0, rows1, acc,
         gsem0, gsem1, ssem0, ssem1, wsem0, wsem1, wssem0, wssem1) = rest
        gbuf = (gbuf0, gbuf1)
        dbuf = (dbuf0, dbuf1)
        wibuf = (wibuf0, wibuf1)
        wbuf = (wbuf0, wbuf1)
        rows = (rows0, rows1)
        gsem = (gsem0, gsem1)
        ssem = (ssem0, ssem1)
        wsem = (wsem0, wsem1)
        wssem = (wssem0, wssem1)
        cid = lax.axis_index("c")
        sid = lax.axis_index("s")
        wid = cid * NS + sid
        ebase = wid * EPW
        zero16 = jnp.zeros((16,), jnp.float32)

        def zfill(i, _):
            for v in range(D // 16):
                rows1[i, pl.ds(v * 16, 16)] = zero16
            return 0
        lax.fori_loop(0, CHUNK, zfill, 0)
        for t in range(NRPT // CHUNK):
            pltpu.sync_copy(rows1,
                            acc.at[pl.ds(sid * NRPT + t * CHUNK, CHUNK), :])

        pltpu.sync_copy(pk_hbm.at[pl.ds(ebase, EPW)], pall)
        if not emit_w:
            pltpu.sync_copy(w_hbm.at[pl.ds(ebase, EPW)],
                            wall.at[pl.ds(0, EPW)])

        def unpack(c, s):
            off = c * CHUNK
            for g in range(CHUNK // 16):
                sl = pl.ds(g * 16, 16)
                pv = pall[pl.ds(off + g * 16, 16)]
                gbuf[s][sl] = pv & _M16
                d = lax.shift_right_logical(pv, 16) & _M14
                dbuf[s][sl] = d
                if emit_w:
                    wibuf[s][sl] = (lax.shift_right_logical(pv, 30) * N_NODES
                                    + d)

        unpack(0, 0)
        pltpu.async_copy(tbl_hbm.at[gbuf0], rows0, gsem0)
        plsc.subcore_barrier()

        def step(c, b, first):
            nb = 1 - b
            pltpu.make_async_copy(tbl_hbm.at[gbuf[b]], rows[b], gsem[b]).wait()
            if emit_w:
                if not first:
                    @pl.when(c >= 2)
                    def _():
                        pltpu.make_async_copy(
                            wbuf[b].at[pl.ds(0, CHUNK)],
                            w_hbm.at[pl.ds(ebase + (c - 2) * CHUNK, CHUNK)],
                            wssem[b]).wait()
                wdesc = pltpu.async_copy(inv_hbm.at[wibuf[b]],
                                         wbuf[b].at[pl.ds(0, CHUNK)], wsem[b])
            if not first:
                pltpu.make_async_copy(rows[nb], acc.at[dbuf[nb]],
                                      ssem[nb]).wait()

            @pl.when(c + 1 < NCH)
            def _():
                unpack(c + 1, nb)
                pltpu.async_copy(tbl_hbm.at[gbuf[nb]], rows[nb], gsem[nb])
            if emit_w:
                wdesc.wait()

                @plsc.parallel_loop(0, CHUNK, 1, unroll=4)
                def scale(j):
                    w = wbuf[b][pl.ds(j, 16)][0]
                    for v in range(D // 16):
                        s2 = pl.ds(v * 16, 16)
                        rows[b][j, s2] = rows[b][j, s2] * w
                pltpu.async_copy(wbuf[b].at[pl.ds(0, CHUNK)],
                                 w_hbm.at[pl.ds(ebase + c * CHUNK, CHUNK)],
                                 wssem[b])
            else:
                off = c * CHUNK

                @plsc.parallel_loop(0, CHUNK, 1, unroll=4)
                def scale(j):
                    w = wall[pl.ds(off + j, 16)][0]
                    for v in range(D // 16):
                        s2 = pl.ds(v * 16, 16)
                        rows[b][j, s2] = rows[b][j, s2] * w
            pltpu.async_copy(rows[b], acc.at[dbuf[b]], ssem[b], add=True)

        step(0, 0, True)

        def pair(k, _):
            step(2 * k + 1, 1, False)
            step(2 * k + 2, 0, False)
            return 0
        lax.fori_loop(0, (NCH - 1) // 2, pair, 0)
        fb = (NCH - 1) % 2
        pltpu.make_async_copy(rows[fb], acc.at[dbuf[fb]], ssem[fb]).wait()
        if emit_w:
            for c in (NCH - 2, NCH - 1):
                bb = c % 2
                pltpu.make_async_copy(
                    wbuf[bb].at[pl.ds(0, CHUNK)],
                    w_hbm.at[pl.ds(ebase + c * CHUNK, CHUNK)],
                    wssem[bb]).wait()
        plsc.subcore_barrier()
        pltpu.sync_copy(acc.at[pl.ds(sid * NRPT, NRPT), :],
                        out_hbm.at[cid, pl.ds(sid * NRPT, NRPT), :])
    return msg_kernel


_msg_emit = _make_msg_kernel(True)
_msg_load = _make_msg_kernel(False)


def kernel(x, edge_index, edge_type, W1, root1, b1, W2, root2, b2):
    x = x.astype(jnp.float32)
    ei = edge_index.astype(jnp.int32)
    et = edge_type.astype(jnp.int32)
    src = ei[0]
    dst = ei[1]

    w1_aug = jnp.concatenate([W1, root1[None]], axis=0)
    h1 = _mm_l1(x, w1_aug, b1.reshape(1, 128))
    packed, cnt = _count_kernel(src, dst, et)
    inv = _inv_counts(cnt)

    m1, w_edge = _msg_emit(h1.reshape((N_REL + 1) * N_NODES, 128), packed,
                           inv)

    w2mask = jnp.zeros((N_REL, 128, 128), jnp.float32)
    for r in range(N_REL):
        w2mask = w2mask.at[r, :, 32 * r:32 * r + 2].set(W2[r])
    root2p = jnp.zeros((128, 16), jnp.float32).at[:, :2].set(root2)
    b2p = jnp.zeros((1, 16), jnp.float32).at[0, :2].set(b2)

    t2, r2 = _mm_l2(h1, m1, w2mask, root2p, b2p)

    m2 = _msg_load(t2.reshape(N_REL * N_NODES, 128), packed, w_edge)

    out16 = _final_add(r2, m2)
    return out16[:, :2]
